# Initial kernel scaffold; baseline (speedup 1.0000x reference)
import jax
import jax.numpy as jnp
from jax.experimental import pallas as pl


def kernel(x_n_cat, tables):
    raise NotImplementedError("write your pallas kernel here")


# speedup vs baseline: 1.0551x; 1.0551x over previous
import functools

import jax
import jax.numpy as jnp
from jax import lax
from jax.experimental import pallas as pl
from jax.experimental.pallas import tpu as pltpu
from jax.experimental.pallas import tpu_sc as plsc

NUM_FIELDS = 26
VOCAB = 100000
HIDDEN = 64
BATCH = 16384

NC, NS = 2, 16
NW = NC * NS
TOTAL_ROWS = BATCH * NUM_FIELDS
ROWS_PER_W = TOTAL_ROWS // NW
CHUNK = 128
NCHUNK = ROWS_PER_W // CHUNK


@functools.partial(
    pl.kernel,
    out_type=jax.ShapeDtypeStruct((TOTAL_ROWS, HIDDEN), jnp.float32),
    mesh=plsc.VectorSubcoreMesh(core_axis_name="c", subcore_axis_name="s"),
    scratch_types=[
        pltpu.VMEM((NCHUNK, CHUNK), jnp.int32),
        pltpu.VMEM((CHUNK, HIDDEN), jnp.float32),
        pltpu.SemaphoreType.DMA,
    ],
    compiler_params=pltpu.CompilerParams(use_tc_tiling_on_sc=False),
)
def _gather_kernel(idx_hbm, table_hbm, out_hbm, idx_v, rows_v, sem):
    wid = lax.axis_index("s") * NC + lax.axis_index("c")
    base = wid * ROWS_PER_W
    pltpu.sync_copy(idx_hbm.at[wid], idx_v)

    def body(j, carry):
        pltpu.async_copy(table_hbm.at[idx_v.at[j]], rows_v, sem).wait()
        pltpu.sync_copy(rows_v, out_hbm.at[pl.ds(base + j * CHUNK, CHUNK)])
        return carry

    lax.fori_loop(0, NCHUNK, body, 0)


def kernel(x_n_cat, tables):
    offsets = (jnp.arange(NUM_FIELDS, dtype=jnp.int32) * VOCAB)[None, :]
    idx = (x_n_cat + offsets).reshape(NW, NCHUNK, CHUNK)
    table = tables.reshape(NUM_FIELDS * VOCAB, HIDDEN)
    out = _gather_kernel(idx, table)
    return out.reshape(BATCH, NUM_FIELDS * HIDDEN)


# speedup vs baseline: 1.1029x; 1.0453x over previous
import functools

import jax
import jax.numpy as jnp
from jax import lax
from jax.experimental import pallas as pl
from jax.experimental.pallas import tpu as pltpu
from jax.experimental.pallas import tpu_sc as plsc

NUM_FIELDS = 26
VOCAB = 100000
HIDDEN = 64
BATCH = 16384

NC, NS = 2, 16
NW = NC * NS
TOTAL_ROWS = BATCH * NUM_FIELDS
ROWS_PER_W = TOTAL_ROWS // NW
CHUNK = 128
NCHUNK = ROWS_PER_W // CHUNK


NBUF = 8


@functools.partial(
    pl.kernel,
    out_type=jax.ShapeDtypeStruct((TOTAL_ROWS, HIDDEN), jnp.float32),
    mesh=plsc.VectorSubcoreMesh(core_axis_name="c", subcore_axis_name="s"),
    scratch_types=[
        pltpu.VMEM((NCHUNK, CHUNK), jnp.int32),
        pltpu.VMEM((NBUF, CHUNK, HIDDEN), jnp.float32),
        pltpu.SemaphoreType.DMA((NBUF,)),
        pltpu.SemaphoreType.DMA((NBUF,)),
    ],
    compiler_params=pltpu.CompilerParams(use_tc_tiling_on_sc=False),
)
def _gather_kernel(idx_hbm, table_hbm, out_hbm, idx_v, bufs, gsem, wsem):
    wid = lax.axis_index("s") * NC + lax.axis_index("c")
    base = wid * ROWS_PER_W
    pltpu.sync_copy(idx_hbm.at[wid], idx_v)

    def gather(j, b):
        return pltpu.make_async_copy(
            table_hbm.at[idx_v.at[j]], bufs.at[b], gsem.at[b])

    def write(j, b):
        return pltpu.make_async_copy(
            bufs.at[b], out_hbm.at[pl.ds(base + j * CHUNK, CHUNK)], wsem.at[b])

    for b in range(NBUF):
        gather(b, b).start()

    @pl.loop(0, NCHUNK - NBUF, step=NBUF)
    def _group(j0):
        for b in range(NBUF):
            gather(j0 + b, b).wait()
            write(j0 + b, b).start()
        for b in range(NBUF):
            write(j0 + b, b).wait()
            gather(j0 + b + NBUF, b).start()

    j0 = NCHUNK - NBUF
    for b in range(NBUF):
        gather(j0 + b, b).wait()
        write(j0 + b, b).start()
    for b in range(NBUF):
        write(j0 + b, b).wait()


def kernel(x_n_cat, tables):
    offsets = (jnp.arange(NUM_FIELDS, dtype=jnp.int32) * VOCAB)[None, :]
    idx = (x_n_cat + offsets).reshape(NW, NCHUNK, CHUNK)
    table = tables.reshape(NUM_FIELDS * VOCAB, HIDDEN)
    out = _gather_kernel(idx, table)
    return out.reshape(BATCH, NUM_FIELDS * HIDDEN)
